# baseline (speedup 1.0000x reference)
import jax
import jax.numpy as jnp
from jax import lax
from jax.experimental import pallas as pl
from jax.experimental.pallas import tpu as pltpu
from jax.experimental.pallas import tpu_sc as plsc

NC, NS, L = 2, 16, 16
NW = NC * NS
BLK = 128
CB = 16
B = CB * BLK
GPB = BLK // L
ROWW = 8


def _build(E, N):
    assert E % B == 0
    nblk = E // BLK
    chunks = E // B
    T = (chunks + NW - 1) // NW
    T2 = (T + 1) // 2

    mesh = plsc.VectorSubcoreMesh(core_axis_name="c", subcore_axis_name="s")

    def body(pos_hbm, bond_hbm, pbc_hbm, lat_hbm, out_hbm,
             bond0, bond1, posi0, posj0, posi1, posj1,
             pbc0, pbc1, o0, o1, lat_v,
             sb0, sb1, sg0, sg1, sp0, sp1, sw0, sw1):
        bondb, posib, posjb = [bond0, bond1], [posi0, posi1], [posj0, posj1]
        pbcb, outb = [pbc0, pbc1], [o0, o1]
        sb, sg, sp, sw = [sb0, sb1], [sg0, sg1], [sp0, sp1], [sw0, sw1]

        wid = lax.axis_index("s") * NC + lax.axis_index("c")
        iota = lax.iota(jnp.int32, L)
        pltpu.sync_copy(lat_hbm, lat_v)
        lb = [[plsc.load_gather(lat_v, [(3 * k + d) * L + iota])
               for d in range(3)] for k in range(3)]
        dvecs = [jnp.full((L,), d, jnp.int32) for d in range(3)]

        def bond_sl(c):
            return bond_hbm.at[pl.ds(c * 2 * CB, 2 * CB)]

        def pbc_sl(c):
            return pbc_hbm.at[pl.ds(c * 3 * CB, 3 * CB)]

        def out_sl(c):
            return out_hbm.at[pl.ds(c * 3 * CB, 3 * CB)]

        def fire_bond(c, p):
            @pl.when(c < chunks)
            def _():
                pltpu.async_copy(bond_sl(c), bondb[p], sb[p])

        def wait_bond(c, p):
            @pl.when(c < chunks)
            def _():
                pltpu.make_async_copy(bond_sl(c), bondb[p], sb[p]).wait()

        def fire_pbc(c, p):
            @pl.when(c < chunks)
            def _():
                pltpu.async_copy(pbc_sl(c), pbcb[p], sp[p])

        def wait_pbc(c, p):
            @pl.when(c < chunks)
            def _():
                pltpu.make_async_copy(pbc_sl(c), pbcb[p], sp[p]).wait()

        def fire_gathers(c, p):
            @pl.when(c < chunks)
            def _():
                for b in range(CB):
                    sl = pl.ds(b * BLK, BLK)
                    pltpu.async_copy(
                        pos_hbm.at[bondb[p].at[2 * b]], posib[p].at[sl], sg[p])
                    pltpu.async_copy(
                        pos_hbm.at[bondb[p].at[2 * b + 1]], posjb[p].at[sl], sg[p])

        def wait_gathers(c, p):
            @pl.when(c < chunks)
            def _():
                for b in range(CB):
                    sl = pl.ds(b * BLK, BLK)
                    pltpu.make_async_copy(
                        pos_hbm.at[bondb[p].at[2 * b]], posib[p].at[sl], sg[p]).wait()
                    pltpu.make_async_copy(
                        pos_hbm.at[bondb[p].at[2 * b + 1]], posjb[p].at[sl], sg[p]).wait()

        def fire_write(c, p):
            @pl.when(c < chunks)
            def _():
                pltpu.async_copy(outb[p], out_sl(c), sw[p])

        def wait_write(c, p):
            @pl.when((c >= 0) & (c < chunks))
            def _():
                pltpu.make_async_copy(outb[p], out_sl(c), sw[p]).wait()

        def compute(c, p):
            @pl.when(c < chunks)
            def _():
                pbc_v, posi, posj, out_v = pbcb[p], posib[p], posjb[p], outb[p]

                def blk_body(g, cr):
                    r = 3 * g
                    for u in range(GPB):
                        ev = g * BLK + u * L + iota
                        pk = [pbc_v[r + k, pl.ds(u * L, L)] for k in range(3)]
                        for d in range(3):
                            pi = plsc.load_gather(posi, [ev, dvecs[d]])
                            pj = plsc.load_gather(posj, [ev, dvecs[d]])
                            off = (pk[0] * lb[0][d] + pk[1] * lb[1][d]
                                   + pk[2] * lb[2][d])
                            out_v[r + d, pl.ds(u * L, L)] = pj - pi + off
                    return cr

                lax.fori_loop(0, CB, blk_body, 0)

        fire_bond(wid, 0)
        fire_bond(NW + wid, 1)
        fire_pbc(wid, 0)
        wait_bond(wid, 0)
        fire_gathers(wid, 0)

        def pair(i, carry):
            for half in range(2):
                k = 2 * i + half
                p, q = half, 1 - half
                c0 = k * NW + wid
                c1 = c0 + NW
                c2 = c1 + NW
                cm2 = c0 - 2 * NW
                wait_gathers(c0, p)
                wait_bond(c1, q)
                fire_gathers(c1, q)
                fire_bond(c2, p)
                wait_pbc(c0, p)
                wait_write(cm2, p)
                compute(c0, p)
                fire_write(c0, p)
                fire_pbc(c1, q)
            return carry

        lax.fori_loop(0, T2, pair, 0)

        for k in (2 * T2 - 2, 2 * T2 - 1):
            wait_write(k * NW + wid, k % 2)

    return pl.kernel(
        body,
        out_type=jax.ShapeDtypeStruct((3 * nblk, BLK), jnp.float32),
        mesh=mesh,
        compiler_params=pltpu.CompilerParams(
            needs_layout_passes=False, use_tc_tiling_on_sc=False),
        scratch_types=[
            pltpu.VMEM((2 * CB, BLK), jnp.int32),
            pltpu.VMEM((2 * CB, BLK), jnp.int32),
            pltpu.VMEM((B, ROWW), jnp.float32),
            pltpu.VMEM((B, ROWW), jnp.float32),
            pltpu.VMEM((B, ROWW), jnp.float32),
            pltpu.VMEM((B, ROWW), jnp.float32),
            pltpu.VMEM((3 * CB, BLK), jnp.float32),
            pltpu.VMEM((3 * CB, BLK), jnp.float32),
            pltpu.VMEM((3 * CB, BLK), jnp.float32),
            pltpu.VMEM((3 * CB, BLK), jnp.float32),
            pltpu.VMEM((9 * L,), jnp.float32),
            pltpu.SemaphoreType.DMA,
            pltpu.SemaphoreType.DMA,
            pltpu.SemaphoreType.DMA,
            pltpu.SemaphoreType.DMA,
            pltpu.SemaphoreType.DMA,
            pltpu.SemaphoreType.DMA,
            pltpu.SemaphoreType.DMA,
            pltpu.SemaphoreType.DMA,
        ],
    )


def kernel(atom_positions, bond_atom_indices, pbc_offsets, lattice):
    E = bond_atom_indices.shape[0]
    N = atom_positions.shape[0]
    nblk = E // BLK
    pos8 = jnp.pad(atom_positions, ((0, 0), (0, ROWW - 3)))
    bond2 = (bond_atom_indices.astype(jnp.int32)
             .reshape(nblk, BLK, 2).transpose(0, 2, 1).reshape(2 * nblk, BLK))
    pbc2 = pbc_offsets.reshape(nblk, BLK, 3).transpose(0, 2, 1).reshape(3 * nblk, BLK)
    latb = jnp.broadcast_to(lattice.reshape(9)[:, None], (9, L)).reshape(9 * L)
    fn = _build(E, N)
    out2 = fn(pos8, bond2, pbc2, latb)
    return out2.reshape(nblk, 3, BLK).transpose(0, 2, 1).reshape(E, 3)


# speedup vs baseline: 1.0519x; 1.0519x over previous
import jax
import jax.numpy as jnp
from jax import lax
from jax.experimental import pallas as pl
from jax.experimental.pallas import tpu as pltpu
from jax.experimental.pallas import tpu_sc as plsc

NC, NS, L = 2, 16, 16
NW = NC * NS
BLK = 128
CB = 16
B = CB * BLK
GPB = BLK // L
ROWW = 8


def _build(E, N):
    assert E % B == 0
    nblk = E // BLK
    chunks = E // B
    T = (chunks + NW - 1) // NW
    T2 = (T + 1) // 2

    mesh = plsc.VectorSubcoreMesh(core_axis_name="c", subcore_axis_name="s")

    def body(pos_hbm, bond_hbm, pbc_hbm, lat_hbm, out_hbm,
             bond0, bond1, posi0, posj0, posi1, posj1,
             pbc0, pbc1, o0, o1, lat_v,
             sb0, sb1, sg0, sg1, sp0, sp1, sw0, sw1):
        bondb, posib, posjb = [bond0, bond1], [posi0, posi1], [posj0, posj1]
        pbcb, outb = [pbc0, pbc1], [o0, o1]
        sb, sg, sp, sw = [sb0, sb1], [sg0, sg1], [sp0, sp1], [sw0, sw1]

        wid = lax.axis_index("s") * NC + lax.axis_index("c")
        iota = lax.iota(jnp.int32, L)
        pltpu.sync_copy(lat_hbm, lat_v)
        lb = [[plsc.load_gather(lat_v, [(3 * k + d) * L + iota])
               for d in range(3)] for k in range(3)]
        dvecs = [jnp.full((L,), d, jnp.int32) for d in range(3)]

        def bond_sl(c):
            return bond_hbm.at[pl.ds(c * 2 * CB, 2 * CB)]

        def pbc_sl(c):
            return pbc_hbm.at[pl.ds(c * 3 * CB, 3 * CB)]

        def out_sl(c):
            return out_hbm.at[pl.ds(c * 4 * CB, 4 * CB)]

        def fire_bond(c, p):
            @pl.when(c < chunks)
            def _():
                pltpu.async_copy(bond_sl(c), bondb[p], sb[p])

        def wait_bond(c, p):
            @pl.when(c < chunks)
            def _():
                pltpu.make_async_copy(bond_sl(c), bondb[p], sb[p]).wait()

        def fire_pbc(c, p):
            @pl.when(c < chunks)
            def _():
                pltpu.async_copy(pbc_sl(c), pbcb[p], sp[p])

        def wait_pbc(c, p):
            @pl.when(c < chunks)
            def _():
                pltpu.make_async_copy(pbc_sl(c), pbcb[p], sp[p]).wait()

        def fire_gathers(c, p):
            @pl.when(c < chunks)
            def _():
                for b in range(CB):
                    sl = pl.ds(b * BLK, BLK)
                    pltpu.async_copy(
                        pos_hbm.at[bondb[p].at[2 * b]], posib[p].at[sl], sg[p])
                    pltpu.async_copy(
                        pos_hbm.at[bondb[p].at[2 * b + 1]], posjb[p].at[sl], sg[p])

        def wait_gathers(c, p):
            @pl.when(c < chunks)
            def _():
                for b in range(CB):
                    sl = pl.ds(b * BLK, BLK)
                    pltpu.make_async_copy(
                        pos_hbm.at[bondb[p].at[2 * b]], posib[p].at[sl], sg[p]).wait()
                    pltpu.make_async_copy(
                        pos_hbm.at[bondb[p].at[2 * b + 1]], posjb[p].at[sl], sg[p]).wait()

        def fire_write(c, p):
            @pl.when(c < chunks)
            def _():
                pltpu.async_copy(outb[p], out_sl(c), sw[p])

        def wait_write(c, p):
            @pl.when((c >= 0) & (c < chunks))
            def _():
                pltpu.make_async_copy(outb[p], out_sl(c), sw[p]).wait()

        def compute(c, p):
            @pl.when(c < chunks)
            def _():
                pbc_v, posi, posj, out_v = pbcb[p], posib[p], posjb[p], outb[p]

                def blk_body(g, cr):
                    r = 4 * g
                    for u in range(GPB):
                        ev = g * BLK + u * L + iota
                        pk = [pbc_v[r + k, pl.ds(u * L, L)] for k in range(3)]
                        for d in range(3):
                            pi = plsc.load_gather(posi, [ev, dvecs[d]])
                            pj = plsc.load_gather(posj, [ev, dvecs[d]])
                            off = (pk[0] * lb[0][d] + pk[1] * lb[1][d]
                                   + pk[2] * lb[2][d])
                            out_v[r + d, pl.ds(u * L, L)] = pj - pi + off
                    return cr

                lax.fori_loop(0, CB, blk_body, 0)

        fire_bond(wid, 0)
        fire_bond(NW + wid, 1)
        fire_pbc(wid, 0)
        wait_bond(wid, 0)
        fire_gathers(wid, 0)

        def pair(i, carry):
            for half in range(2):
                k = 2 * i + half
                p, q = half, 1 - half
                c0 = k * NW + wid
                c1 = c0 + NW
                c2 = c1 + NW
                cm2 = c0 - 2 * NW
                wait_gathers(c0, p)
                wait_bond(c1, q)
                fire_gathers(c1, q)
                fire_bond(c2, p)
                wait_pbc(c0, p)
                wait_write(cm2, p)
                compute(c0, p)
                fire_write(c0, p)
                fire_pbc(c1, q)
            return carry

        lax.fori_loop(0, T2, pair, 0)

        for k in (2 * T2 - 2, 2 * T2 - 1):
            wait_write(k * NW + wid, k % 2)

    return pl.kernel(
        body,
        out_type=jax.ShapeDtypeStruct((4 * nblk, BLK), jnp.float32),
        mesh=mesh,
        compiler_params=pltpu.CompilerParams(
            needs_layout_passes=False, use_tc_tiling_on_sc=False),
        scratch_types=[
            pltpu.VMEM((2 * CB, BLK), jnp.int32),
            pltpu.VMEM((2 * CB, BLK), jnp.int32),
            pltpu.VMEM((B, ROWW), jnp.float32),
            pltpu.VMEM((B, ROWW), jnp.float32),
            pltpu.VMEM((B, ROWW), jnp.float32),
            pltpu.VMEM((B, ROWW), jnp.float32),
            pltpu.VMEM((3 * CB, BLK), jnp.float32),
            pltpu.VMEM((3 * CB, BLK), jnp.float32),
            pltpu.VMEM((4 * CB, BLK), jnp.float32),
            pltpu.VMEM((4 * CB, BLK), jnp.float32),
            pltpu.VMEM((9 * L,), jnp.float32),
            pltpu.SemaphoreType.DMA,
            pltpu.SemaphoreType.DMA,
            pltpu.SemaphoreType.DMA,
            pltpu.SemaphoreType.DMA,
            pltpu.SemaphoreType.DMA,
            pltpu.SemaphoreType.DMA,
            pltpu.SemaphoreType.DMA,
            pltpu.SemaphoreType.DMA,
        ],
    )


def kernel(atom_positions, bond_atom_indices, pbc_offsets, lattice):
    E = bond_atom_indices.shape[0]
    N = atom_positions.shape[0]
    nblk = E // BLK
    pos8 = jnp.pad(atom_positions, ((0, 0), (0, ROWW - 3)))
    bond2 = (bond_atom_indices.astype(jnp.int32)
             .reshape(nblk, BLK, 2).transpose(0, 2, 1).reshape(2 * nblk, BLK))
    pbc2 = pbc_offsets.reshape(nblk, BLK, 3).transpose(0, 2, 1).reshape(3 * nblk, BLK)
    latb = jnp.broadcast_to(lattice.reshape(9)[:, None], (9, L)).reshape(9 * L)
    fn = _build(E, N)
    out4 = fn(pos8, bond2, pbc2, latb)
    return (out4.reshape(nblk, 4, BLK)[:, :3, :]
            .transpose(0, 2, 1).reshape(E, 3))


# speedup vs baseline: 1.0560x; 1.0040x over previous
import jax
import jax.numpy as jnp
from jax import lax
from jax.experimental import pallas as pl
from jax.experimental.pallas import tpu as pltpu
from jax.experimental.pallas import tpu_sc as plsc

NC, NS, L = 2, 16, 16
NW = NC * NS
BLK = 128
CB = 16
B = CB * BLK
GPB = BLK // L
ROWW = 8


def _build(E, N):
    assert E % B == 0
    nblk = E // BLK
    chunks = E // B
    T = (chunks + NW - 1) // NW
    T2 = (T + 1) // 2

    mesh = plsc.VectorSubcoreMesh(core_axis_name="c", subcore_axis_name="s")

    def body(pos_hbm, bond_hbm, pbc_hbm, lat_hbm, out_hbm,
             bond0, bond1, posi0, posj0, posi1, posj1,
             pbc0, pbc1, o0, o1, lat_v,
             sb0, sb1, sg0, sg1, sp0, sp1, sw0, sw1):
        bondb, posib, posjb = [bond0, bond1], [posi0, posi1], [posj0, posj1]
        pbcb, outb = [pbc0, pbc1], [o0, o1]
        sb, sg, sp, sw = [sb0, sb1], [sg0, sg1], [sp0, sp1], [sw0, sw1]

        wid = lax.axis_index("s") * NC + lax.axis_index("c")
        iota = lax.iota(jnp.int32, L)
        pltpu.sync_copy(lat_hbm, lat_v)
        lb = [[plsc.load_gather(lat_v, [(3 * k + d) * L + iota])
               for d in range(3)] for k in range(3)]
        dvecs = [jnp.full((L,), d, jnp.int32) for d in range(3)]

        def bond_sl(c):
            return bond_hbm.at[pl.ds(c * 2 * CB, 2 * CB)]

        def pbc_sl(c):
            return pbc_hbm.at[pl.ds(c * 3 * CB, 3 * CB)]

        def out_sl(c):
            return out_hbm.at[pl.ds(c * 4 * CB, 4 * CB)]

        def fire_bond(c, p):
            @pl.when(c < chunks)
            def _():
                pltpu.async_copy(bond_sl(c), bondb[p], sb[p])

        def wait_bond(c, p):
            @pl.when(c < chunks)
            def _():
                pltpu.make_async_copy(bond_sl(c), bondb[p], sb[p]).wait()

        def fire_pbc(c, p):
            @pl.when(c < chunks)
            def _():
                pltpu.async_copy(pbc_sl(c), pbcb[p], sp[p])

        def wait_pbc(c, p):
            @pl.when(c < chunks)
            def _():
                pltpu.make_async_copy(pbc_sl(c), pbcb[p], sp[p]).wait()

        def fire_gathers(c, p):
            @pl.when(c < chunks)
            def _():
                for b in range(CB):
                    sl = pl.ds(b * BLK, BLK)
                    pltpu.async_copy(
                        pos_hbm.at[bondb[p].at[2 * b]], posib[p].at[sl], sg[p])
                    pltpu.async_copy(
                        pos_hbm.at[bondb[p].at[2 * b + 1]], posjb[p].at[sl], sg[p])

        def wait_gathers(c, p):
            @pl.when(c < chunks)
            def _():
                for b in range(CB):
                    sl = pl.ds(b * BLK, BLK)
                    pltpu.make_async_copy(
                        pos_hbm.at[bondb[p].at[2 * b]], posib[p].at[sl], sg[p]).wait()
                    pltpu.make_async_copy(
                        pos_hbm.at[bondb[p].at[2 * b + 1]], posjb[p].at[sl], sg[p]).wait()

        def fire_write(c, p):
            @pl.when(c < chunks)
            def _():
                pltpu.async_copy(outb[p], out_sl(c), sw[p])

        def wait_write(c, p):
            @pl.when((c >= 0) & (c < chunks))
            def _():
                pltpu.make_async_copy(outb[p], out_sl(c), sw[p]).wait()

        def compute(c, p):
            @pl.when(c < chunks)
            def _():
                pbc_v, posi, posj, out_v = pbcb[p], posib[p], posjb[p], outb[p]

                def blk_body(g, cr):
                    rp = 3 * g
                    ro = 4 * g
                    for u in range(GPB):
                        ev = g * BLK + u * L + iota
                        pk = [pbc_v[rp + k, pl.ds(u * L, L)] for k in range(3)]
                        for d in range(3):
                            pi = plsc.load_gather(posi, [ev, dvecs[d]])
                            pj = plsc.load_gather(posj, [ev, dvecs[d]])
                            off = (pk[0] * lb[0][d] + pk[1] * lb[1][d]
                                   + pk[2] * lb[2][d])
                            out_v[ro + d, pl.ds(u * L, L)] = pj - pi + off
                    return cr

                lax.fori_loop(0, CB, blk_body, 0)

        fire_bond(wid, 0)
        fire_bond(NW + wid, 1)
        fire_pbc(wid, 0)
        wait_bond(wid, 0)
        fire_gathers(wid, 0)

        def pair(i, carry):
            for half in range(2):
                k = 2 * i + half
                p, q = half, 1 - half
                c0 = k * NW + wid
                c1 = c0 + NW
                c2 = c1 + NW
                cm2 = c0 - 2 * NW
                wait_gathers(c0, p)
                wait_bond(c1, q)
                fire_gathers(c1, q)
                fire_bond(c2, p)
                wait_pbc(c0, p)
                wait_write(cm2, p)
                compute(c0, p)
                fire_write(c0, p)
                fire_pbc(c1, q)
            return carry

        lax.fori_loop(0, T2, pair, 0)

        for k in (2 * T2 - 2, 2 * T2 - 1):
            wait_write(k * NW + wid, k % 2)

    return pl.kernel(
        body,
        out_type=jax.ShapeDtypeStruct((4 * nblk, BLK), jnp.float32),
        mesh=mesh,
        compiler_params=pltpu.CompilerParams(
            needs_layout_passes=False, use_tc_tiling_on_sc=False),
        scratch_types=[
            pltpu.VMEM((2 * CB, BLK), jnp.int32),
            pltpu.VMEM((2 * CB, BLK), jnp.int32),
            pltpu.VMEM((B, ROWW), jnp.float32),
            pltpu.VMEM((B, ROWW), jnp.float32),
            pltpu.VMEM((B, ROWW), jnp.float32),
            pltpu.VMEM((B, ROWW), jnp.float32),
            pltpu.VMEM((3 * CB, BLK), jnp.float32),
            pltpu.VMEM((3 * CB, BLK), jnp.float32),
            pltpu.VMEM((4 * CB, BLK), jnp.float32),
            pltpu.VMEM((4 * CB, BLK), jnp.float32),
            pltpu.VMEM((9 * L,), jnp.float32),
            pltpu.SemaphoreType.DMA,
            pltpu.SemaphoreType.DMA,
            pltpu.SemaphoreType.DMA,
            pltpu.SemaphoreType.DMA,
            pltpu.SemaphoreType.DMA,
            pltpu.SemaphoreType.DMA,
            pltpu.SemaphoreType.DMA,
            pltpu.SemaphoreType.DMA,
        ],
    )


def kernel(atom_positions, bond_atom_indices, pbc_offsets, lattice):
    E = bond_atom_indices.shape[0]
    N = atom_positions.shape[0]
    nblk = E // BLK
    pos8 = jnp.pad(atom_positions, ((0, 0), (0, ROWW - 3)))
    bond2 = (bond_atom_indices.astype(jnp.int32)
             .reshape(nblk, BLK, 2).transpose(0, 2, 1).reshape(2 * nblk, BLK))
    pbc2 = pbc_offsets.reshape(nblk, BLK, 3).transpose(0, 2, 1).reshape(3 * nblk, BLK)
    latb = jnp.broadcast_to(lattice.reshape(9)[:, None], (9, L)).reshape(9 * L)
    fn = _build(E, N)
    out4 = fn(pos8, bond2, pbc2, latb)
    return (out4.reshape(nblk, 4, BLK)[:, :3, :]
            .transpose(0, 2, 1).reshape(E, 3))
